# baseline (speedup 1.0000x reference)
import functools

import jax
import jax.numpy as jnp
from jax import lax
from jax.experimental import pallas as pl
from jax.experimental.pallas import tpu as pltpu
from jax.experimental.pallas import tpu_sc as plsc

N_NODES = 10000
N_EDGES = 320000
D = 128
DH = 64

NC = 2
NS = 16
EDGES_PER_TILE = N_EDGES // NS
CHUNK = 125
NCHUNK = EDGES_PER_TILE // CHUNK
ROWS_PER_TILE = N_NODES // NS
HROWS = N_NODES // 16
HSTAGE = EDGES_PER_TILE // 16

_sc_mesh = plsc.VectorSubcoreMesh(
    core_axis_name="c", subcore_axis_name="s", num_cores=NC, num_subcores=NS
)


@functools.partial(
    pl.kernel,
    out_type=(
        jax.ShapeDtypeStruct((NC, N_NODES, DH), jnp.float32),
        jax.ShapeDtypeStruct((HROWS, 16), jnp.float32),
    ),
    mesh=_sc_mesh,
    scratch_types=[
        pltpu.VMEM((NCHUNK, CHUNK), jnp.int32),
        pltpu.VMEM((NCHUNK, CHUNK), jnp.int32),
        pltpu.VMEM((HSTAGE, 16), jnp.int32),
        pltpu.VMEM((2, CHUNK, DH), jnp.float32),
        pltpu.VMEM((HROWS, 16), jnp.float32),
        pltpu.VMEM((5, CHUNK), jnp.int32),
        pltpu.VMEM_SHARED((N_NODES, DH), jnp.float32),
        pltpu.VMEM_SHARED((HROWS, 16), jnp.float32),
        pltpu.SemaphoreType.DMA,
        pltpu.SemaphoreType.DMA,
        pltpu.SemaphoreType.DMA,
        pltpu.SemaphoreType.DMA,
    ],
    compiler_params=pltpu.CompilerParams(
        use_tc_tiling_on_sc=False, needs_layout_passes=False
    ),
)
def _aggregate(xh_hbm, edges_hbm, dstf_hbm, zrows_hbm, zdeg_hbm, iota_hbm,
               outf_hbm, outd_hbm,
               src_v, dst_v, dstf_v, rows_v, hist_v, iota_v,
               acc_sh, deg_sh, gsem0, gsem1, ssem0, ssem1):
    cid = lax.axis_index("c")
    sid = lax.axis_index("s")
    xp = xh_hbm.at[cid]

    pltpu.sync_copy(edges_hbm.at[0, sid], src_v)
    pltpu.sync_copy(edges_hbm.at[1, sid], dst_v)
    pltpu.sync_copy(dstf_hbm.at[sid], dstf_v)
    pltpu.sync_copy(iota_hbm, iota_v)

    pltpu.sync_copy(zrows_hbm, acc_sh.at[pl.ds(sid * ROWS_PER_TILE, ROWS_PER_TILE)])
    pltpu.sync_copy(zdeg_hbm, hist_v)

    @pl.when(sid == 0)
    def _():
        pltpu.sync_copy(zdeg_hbm, deg_sh)

    plsc.subcore_barrier()

    gsems = (gsem0, gsem1)
    ssems = (ssem0, ssem1)
    ones16 = jnp.full((16,), 1.0, jnp.float32)

    pltpu.async_copy(xp.at[src_v.at[0]], rows_v.at[0], gsem0)
    pltpu.async_copy(xp.at[src_v.at[1]], rows_v.at[1], gsem1)

    def hist_slice(lo, hi):
        def hbody(i, _):
            d = dstf_v[i]
            plsc.addupdate_scatter(hist_v, [d >> 4, d & 15], ones16)
            return 0
        lax.fori_loop(lo, hi, hbody, 0)

    def body(jj, _):
        for b in (0, 1):
            t = 2 * jj + b
            pltpu.make_async_copy(xp.at[src_v.at[t]], rows_v.at[b], gsems[b]).wait()
            d = pltpu.async_copy(
                rows_v.at[b], acc_sh.at[dst_v.at[t]], ssems[b], add=True
            )
            hist_slice((HSTAGE * t) // NCHUNK, (HSTAGE * (t + 1)) // NCHUNK)
            d.wait()

            @pl.when(t + 2 < NCHUNK)
            def _():
                pltpu.async_copy(xp.at[src_v.at[t + 2]], rows_v.at[b], gsems[b])

        return 0

    lax.fori_loop(0, NCHUNK // 2, body, 0)

    @pl.when(cid == 0)
    def _():
        for c5 in range(5):
            pltpu.sync_copy(
                hist_v.at[pl.ds(c5 * CHUNK, CHUNK)],
                deg_sh.at[iota_v.at[c5]],
                add=True,
            )

    plsc.subcore_barrier()
    row0 = sid * ROWS_PER_TILE
    pltpu.sync_copy(
        acc_sh.at[pl.ds(row0, ROWS_PER_TILE)],
        outf_hbm.at[cid, pl.ds(row0, ROWS_PER_TILE)],
    )

    @pl.when((cid == 0) & (sid == 0))
    def _():
        pltpu.sync_copy(deg_sh, outd_hbm)


ROW_BLK = 2000


def _finish_body(parts_ref, deg_ref, wt0_ref, wt1_ref, b_ref, out_ref):
    inv = 1.0 / jnp.maximum(deg_ref[...], 1.0)
    h0 = parts_ref[0] * inv
    h1 = parts_ref[1] * inv
    out_ref[...] = (
        jnp.dot(h0, wt0_ref[...], preferred_element_type=jnp.float32)
        + jnp.dot(h1, wt1_ref[...], preferred_element_type=jnp.float32)
        + b_ref[...] + 0.01
    )


_finish = pl.pallas_call(
    _finish_body,
    grid=(N_NODES // ROW_BLK,),
    in_specs=[
        pl.BlockSpec((NC, ROW_BLK, DH), lambda i: (0, i, 0)),
        pl.BlockSpec((ROW_BLK, 1), lambda i: (i, 0)),
        pl.BlockSpec((DH, D), lambda i: (0, 0)),
        pl.BlockSpec((DH, D), lambda i: (0, 0)),
        pl.BlockSpec((1, D), lambda i: (0, 0)),
    ],
    out_specs=pl.BlockSpec((ROW_BLK, D), lambda i: (i, 0)),
    out_shape=jax.ShapeDtypeStruct((N_NODES, D), jnp.float32),
)


@jax.jit
def kernel(x, edge_index, W_neigh, b_neigh):
    xh = jnp.stack([x[:, :DH], x[:, DH:]])
    edges = edge_index.reshape(2, NS, NCHUNK, CHUNK)
    dstf = edge_index[1].reshape(NS, HSTAGE, 16)
    zrows = jnp.zeros((ROWS_PER_TILE, DH), jnp.float32)
    zdeg = jnp.zeros((HROWS, 16), jnp.float32)
    iota = jnp.arange(HROWS, dtype=jnp.int32).reshape(5, CHUNK)
    parts, deg = _aggregate(xh, edges, dstf, zrows, zdeg, iota)
    wt = W_neigh.T
    return _finish(
        parts, deg.reshape(N_NODES, 1), wt[:DH], wt[DH:], b_neigh.reshape(1, D)
    )


# speedup vs baseline: 1.0830x; 1.0830x over previous
import functools

import jax
import jax.numpy as jnp
from jax import lax
from jax.experimental import pallas as pl
from jax.experimental.pallas import tpu as pltpu
from jax.experimental.pallas import tpu_sc as plsc

N_NODES = 10000
N_EDGES = 320000
D = 128
DH = 64

NC = 2
NS = 16
EDGES_PER_TILE = N_EDGES // NS
CHUNK = 80
NCHUNK = EDGES_PER_TILE // CHUNK
GPC = CHUNK // 16
NPAD = 10240
RPT = NPAD // NS
HR = NPAD // 16

_sc_mesh = plsc.VectorSubcoreMesh(
    core_axis_name="c", subcore_axis_name="s", num_cores=NC, num_subcores=NS
)


@functools.partial(
    pl.kernel,
    out_type=(
        jax.ShapeDtypeStruct((NC, NPAD, DH), jnp.float32),
        jax.ShapeDtypeStruct((HR, 16), jnp.float32),
    ),
    mesh=_sc_mesh,
    scratch_types=[
        pltpu.VMEM((NCHUNK, CHUNK), jnp.int32),
        pltpu.VMEM((NCHUNK, CHUNK), jnp.int32),
        pltpu.VMEM((2, CHUNK, DH), jnp.float32),
        pltpu.VMEM((HR, 16), jnp.float32),
        pltpu.VMEM((HR // 128, 128), jnp.int32),
        pltpu.VMEM_SHARED((NPAD, DH), jnp.float32),
        pltpu.VMEM_SHARED((HR, 16), jnp.float32),
        pltpu.SemaphoreType.DMA,
        pltpu.SemaphoreType.DMA,
        pltpu.SemaphoreType.DMA,
        pltpu.SemaphoreType.DMA,
    ],
    compiler_params=pltpu.CompilerParams(
        use_tc_tiling_on_sc=False, needs_layout_passes=False
    ),
)
def _aggregate(x_hbm, edges_hbm, zrows_hbm, zhist_hbm, iota_hbm,
               outf_hbm, outd_hbm,
               src_v, dst_v, rows_v, hist_v, iota_v,
               acc_sh, deg_sh, gsem0, gsem1, ssem0, ssem1):
    cid = lax.axis_index("c")
    sid = lax.axis_index("s")

    pltpu.sync_copy(edges_hbm.at[0, sid], src_v)
    pltpu.sync_copy(edges_hbm.at[1, sid], dst_v)
    pltpu.sync_copy(iota_hbm, iota_v)

    pltpu.sync_copy(zrows_hbm, acc_sh.at[pl.ds(sid * RPT, RPT)])
    pltpu.sync_copy(zhist_hbm, hist_v)
    pltpu.sync_copy(
        zhist_hbm.at[pl.ds(0, RPT // 16)],
        deg_sh.at[pl.ds(sid * (RPT // 16), RPT // 16)],
    )
    plsc.subcore_barrier()

    gsems = (gsem0, gsem1)
    ssems = (ssem0, ssem1)
    ones16 = jnp.full((16,), 1.0, jnp.float32)

    def transform_chunk(t):
        for k in range(GPC):
            sl = pl.ds(k * 16, 16)
            s = src_v[t, sl]
            src_v[t, sl] = s + s + cid

    def hist_chunk(t):
        for k in range(GPC):
            d = dst_v[t, pl.ds(k * 16, 16)]
            plsc.addupdate_scatter(hist_v, [d >> 4, d & 15], ones16)

    transform_chunk(0)
    transform_chunk(1)
    pltpu.async_copy(x_hbm.at[src_v.at[0]], rows_v.at[0], gsem0)
    pltpu.async_copy(x_hbm.at[src_v.at[1]], rows_v.at[1], gsem1)

    def body(jj, _):
        for b in (0, 1):
            t = 2 * jj + b
            pltpu.make_async_copy(
                x_hbm.at[src_v.at[t]], rows_v.at[b], gsems[b]
            ).wait()
            d = pltpu.async_copy(
                rows_v.at[b], acc_sh.at[dst_v.at[t]], ssems[b], add=True
            )
            hist_chunk(t)

            @pl.when(t + 2 < NCHUNK)
            def _():
                transform_chunk(t + 2)

            d.wait()

            @pl.when(t + 2 < NCHUNK)
            def _():
                pltpu.async_copy(
                    x_hbm.at[src_v.at[t + 2]], rows_v.at[b], gsems[b]
                )

        return 0

    lax.fori_loop(0, NCHUNK // 2, body, 0)

    for c5 in range(HR // 128):
        pltpu.sync_copy(
            hist_v.at[pl.ds(c5 * 128, 128)],
            deg_sh.at[iota_v.at[c5]],
            add=True,
        )

    plsc.subcore_barrier()

    row0 = sid * RPT
    pltpu.sync_copy(acc_sh.at[pl.ds(row0, RPT)], outf_hbm.at[cid, pl.ds(row0, RPT)])

    @pl.when((cid == 0) & (sid == 0))
    def _():
        pltpu.sync_copy(deg_sh, outd_hbm)


NPAIR = N_NODES // 2
PBLK = 1000


def _finish_body(pv_ref, dv_ref, w_ref, b_ref, out_ref):
    a = pv_ref[0]
    bb = pv_ref[1]
    h_e = jnp.concatenate([a[:, :DH], bb[:, :DH]], axis=1)
    h_o = jnp.concatenate([a[:, DH:], bb[:, DH:]], axis=1)
    inv = 1.0 / jnp.maximum(dv_ref[...], 1.0)
    w = w_ref[...]
    dims = (((1,), (1,)), ((), ()))
    o_e = (
        lax.dot_general(h_e, w, dims, preferred_element_type=jnp.float32)
        * inv[:, 0:1] + b_ref[...] + 0.01
    )
    o_o = (
        lax.dot_general(h_o, w, dims, preferred_element_type=jnp.float32)
        * inv[:, 1:2] + b_ref[...] + 0.01
    )
    out_ref[...] = jnp.concatenate([o_e, o_o], axis=1)


_finish = pl.pallas_call(
    _finish_body,
    grid=(NPAIR // PBLK,),
    in_specs=[
        pl.BlockSpec((NC, PBLK, D), lambda i: (0, i, 0)),
        pl.BlockSpec((PBLK, 2), lambda i: (i, 0)),
        pl.BlockSpec((D, D), lambda i: (0, 0)),
        pl.BlockSpec((1, D), lambda i: (0, 0)),
    ],
    out_specs=pl.BlockSpec((PBLK, 2 * D), lambda i: (i, 0)),
    out_shape=jax.ShapeDtypeStruct((NPAIR, 2 * D), jnp.float32),
)


@jax.jit
def kernel(x, edge_index, W_neigh, b_neigh):
    xv = x.reshape(2 * N_NODES, DH)
    edges = edge_index.reshape(2, NS, NCHUNK, CHUNK)
    zrows = jnp.zeros((RPT, DH), jnp.float32)
    zhist = jnp.zeros((HR, 16), jnp.float32)
    iota = jnp.arange(HR, dtype=jnp.int32).reshape(HR // 128, 128)
    parts, deg = _aggregate(xv, edges, zrows, zhist, iota)
    pv = parts.reshape(NC, NPAD // 2, D)
    dv = deg.reshape(NPAD // 2, 2)[:NPAIR]
    out = _finish(pv, dv, W_neigh, b_neigh.reshape(1, D))
    return out.reshape(N_NODES, D)
